# baseline (speedup 1.0000x reference)
import functools

import jax
import jax.numpy as jnp
from jax import lax
from jax.experimental import pallas as pl
from jax.experimental.pallas import tpu as pltpu
from jax.experimental.pallas import tpu_sc as plsc

LANES = 16
BLK = 128
KPB = 4


def kernel(x, weight):
    batch, hist = x.shape
    vocab, dim = weight.shape

    info = plsc.get_sparse_core_info()
    nw = info.num_cores * info.num_subcores
    assert batch == BLK * KPB * nw and dim % LANES == 0 and hist % 2 == 0

    xt = jnp.transpose(x.astype(jnp.int32))

    o5_rows = hist * (dim // 8) * (batch // BLK)

    mesh = plsc.VectorSubcoreMesh(core_axis_name="c", subcore_axis_name="s")

    @functools.partial(
        pl.kernel,
        out_type=jax.ShapeDtypeStruct((o5_rows, 8, BLK), jnp.float32),
        mesh=mesh,
        compiler_params=pltpu.CompilerParams(
            use_tc_tiling_on_sc=False, needs_layout_passes=False
        ),
        scratch_types=[
            pltpu.VMEM((hist, KPB * BLK), jnp.int32),
            pltpu.VMEM((2 * KPB, BLK, dim), jnp.float32),
            pltpu.VMEM((2, dim, BLK + 1), jnp.float32),
            [pltpu.SemaphoreType.DMA] * (2 * KPB),
            pltpu.SemaphoreType.DMA,
            pltpu.SemaphoreType.DMA,
        ],
    )
    def gather_kernel(
        xt_hbm, table_hbm, o_hbm, idx_v, buf, tbuf, gsems, ssem0, ssem1
    ):
        wid = lax.axis_index("s") * info.num_cores + lax.axis_index("c")
        pltpu.sync_copy(xt_hbm.at[:, pl.ds(wid * KPB * BLK, KPB * BLK)], idx_v)

        ssems = (ssem0, ssem1)
        iota = lax.iota(jnp.int32, LANES)

        def fire_gather(h, k, slot):
            pltpu.async_copy(
                table_hbm.at[idx_v.at[h, pl.ds(k * BLK, BLK)]],
                buf.at[slot],
                gsems[slot],
            )

        def drain_gather(slot):
            pltpu.make_async_copy(
                table_hbm.at[pl.ds(0, BLK)], buf.at[slot], gsems[slot]
            ).wait()

        riota = tuple(iota + k * LANES for k in range(dim // LANES))

        def transpose(sg, st):
            bufs = buf.at[sg]
            tbufs = tbuf.at[st]

            @plsc.parallel_loop(0, BLK, unroll=8)
            def _(j):
                cols = jnp.full((LANES,), j, jnp.int32)
                for k in range(dim // LANES):
                    vals = bufs[j, pl.ds(k * LANES, LANES)]
                    plsc.store_scatter(tbufs, [riota[k], cols], vals)

        def fire_store(h, k, st):
            b_hi = wid * KPB + k
            r0 = (h * (dim // 8)) * BLK + b_hi
            for c_hi in range(dim // 8):
                pltpu.async_copy(
                    tbuf.at[st, pl.ds(c_hi * 8, 8), pl.ds(0, BLK)],
                    o_hbm.at[r0 + c_hi * BLK],
                    ssems[st],
                )

        def wait_store(st):
            for c_hi in range(dim // 8):
                pltpu.make_async_copy(
                    o_hbm.at[0],
                    tbuf.at[st, pl.ds(c_hi * 8, 8), pl.ds(0, BLK)],
                    ssems[st],
                ).wait()

        def body(h2, carry, first=False, last=False):
            h = 2 * h2
            for half in range(2):
                for k in range(KPB):
                    slot = half * KPB + k
                    drain_gather(slot)
                    if not (first and slot < 2):
                        wait_store(k % 2)
                    transpose(slot, k % 2)
                    fire_store(h + half, k, k % 2)
                    if not last:
                        fire_gather(h + half + 2, k, slot)
            return carry

        for half in range(2):
            for k in range(KPB):
                fire_gather(half, k, half * KPB + k)
        body(0, 0, first=True)
        lax.fori_loop(1, hist // 2 - 1, body, 0)
        body(hist // 2 - 1, 0, last=True)
        wait_store(0)
        wait_store(1)

    o5 = gather_kernel(xt, weight)
    o5 = o5.reshape(hist, dim // 8, batch // BLK, 8, BLK)
    out = o5.transpose(2, 4, 0, 1, 3).reshape(batch, hist, dim)
    return out


# speedup vs baseline: 1.0125x; 1.0125x over previous
import functools

import jax
import jax.numpy as jnp
from jax import lax
from jax.experimental import pallas as pl
from jax.experimental.pallas import tpu as pltpu
from jax.experimental.pallas import tpu_sc as plsc

LANES = 16
BLK = 128
KPB = 4


def kernel(x, weight):
    batch, hist = x.shape
    vocab, dim = weight.shape

    info = plsc.get_sparse_core_info()
    nw = info.num_cores * info.num_subcores
    assert batch == BLK * KPB * nw and dim % LANES == 0 and hist % 2 == 0

    xt = jnp.transpose(x.astype(jnp.int32))

    o5_rows = hist * (dim // 8) * (batch // BLK)

    mesh = plsc.VectorSubcoreMesh(core_axis_name="c", subcore_axis_name="s")

    @functools.partial(
        pl.kernel,
        out_type=jax.ShapeDtypeStruct((o5_rows, 8, BLK), jnp.float32),
        mesh=mesh,
        compiler_params=pltpu.CompilerParams(
            use_tc_tiling_on_sc=False, needs_layout_passes=False
        ),
        scratch_types=[
            pltpu.VMEM((hist, KPB * BLK), jnp.int32),
            pltpu.VMEM((KPB, BLK, dim), jnp.float32),
            pltpu.VMEM((2, dim, BLK + 1), jnp.float32),
            pltpu.SemaphoreType.DMA,
            pltpu.SemaphoreType.DMA,
            pltpu.SemaphoreType.DMA,
            pltpu.SemaphoreType.DMA,
            pltpu.SemaphoreType.DMA,
            pltpu.SemaphoreType.DMA,
        ],
    )
    def gather_kernel(
        xt_hbm, table_hbm, o_hbm, idx_v, buf, tbuf,
        gsem0, gsem1, gsem2, gsem3, ssem0, ssem1,
    ):
        wid = lax.axis_index("s") * info.num_cores + lax.axis_index("c")
        pltpu.sync_copy(xt_hbm.at[:, pl.ds(wid * KPB * BLK, KPB * BLK)], idx_v)

        gsems = (gsem0, gsem1, gsem2, gsem3)
        ssems = (ssem0, ssem1)
        iota = lax.iota(jnp.int32, LANES)

        def fire_gather(h, k):
            pltpu.async_copy(
                table_hbm.at[idx_v.at[h, pl.ds(k * BLK, BLK)]],
                buf.at[k],
                gsems[k],
            )

        def drain_gather(k):
            pltpu.make_async_copy(
                table_hbm.at[pl.ds(0, BLK)], buf.at[k], gsems[k]
            ).wait()

        riota = tuple(iota + k * LANES for k in range(dim // LANES))

        def transpose(sg, st):
            bufs = buf.at[sg]
            tbufs = tbuf.at[st]

            @plsc.parallel_loop(0, BLK, unroll=8)
            def _(j):
                cols = jnp.full((LANES,), j, jnp.int32)
                for k in range(dim // LANES):
                    vals = bufs[j, pl.ds(k * LANES, LANES)]
                    plsc.store_scatter(tbufs, [riota[k], cols], vals)

        def fire_store(h, k, st):
            b_hi = wid * KPB + k
            r0 = (h * (dim // 8)) * BLK + b_hi
            for c_hi in range(dim // 8):
                pltpu.async_copy(
                    tbuf.at[st, pl.ds(c_hi * 8, 8), pl.ds(0, BLK)],
                    o_hbm.at[r0 + c_hi * BLK],
                    ssems[st],
                )

        def wait_store(st):
            for c_hi in range(dim // 8):
                pltpu.make_async_copy(
                    o_hbm.at[0],
                    tbuf.at[st, pl.ds(c_hi * 8, 8), pl.ds(0, BLK)],
                    ssems[st],
                ).wait()

        def body(h, carry, first=False, last=False):
            for k in range(KPB):
                drain_gather(k)
                if not (first and k < 2):
                    wait_store(k % 2)
                transpose(k, k % 2)
                fire_store(h, k, k % 2)
                if not last:
                    fire_gather(h + 1, k)
            return carry

        for k in range(KPB):
            fire_gather(0, k)
        body(0, 0, first=True)
        lax.fori_loop(1, hist - 1, body, 0)
        body(hist - 1, 0, last=True)
        wait_store(0)
        wait_store(1)

    o5 = gather_kernel(xt, weight)
    o5 = o5.reshape(hist, dim // 8, batch // BLK, 8, BLK)
    out = o5.transpose(2, 4, 0, 1, 3).reshape(batch, hist, dim)
    return out
